# baseline (speedup 1.0000x reference)
import jax
import jax.numpy as jnp
from jax.experimental import pallas as pl
from jax.experimental.pallas import tpu as pltpu

_NCHUNKS = 8


def _copy_body(in_ref, out_ref, sems):
    d = in_ref.shape[0]
    rows = d // _NCHUNKS if d >= _NCHUNKS else 1
    nch = d // rows
    for c in range(nch):
        pltpu.make_async_copy(
            in_ref.at[pl.ds(c * rows, rows), :],
            out_ref.at[pl.ds(c * rows, rows), :],
            sems.at[c],
        ).start()
    for c in range(nch):
        pltpu.make_async_copy(
            in_ref.at[pl.ds(c * rows, rows), :],
            out_ref.at[pl.ds(c * rows, rows), :],
            sems.at[c],
        ).wait()


def kernel(c_embeddings):
    n, d = c_embeddings.shape
    xt = c_embeddings.T
    out = pl.pallas_call(
        _copy_body,
        out_shape=jax.ShapeDtypeStruct((d, n), xt.dtype),
        in_specs=[pl.BlockSpec(memory_space=pl.ANY)],
        out_specs=pl.BlockSpec(memory_space=pl.ANY),
        scratch_shapes=[pltpu.SemaphoreType.DMA((_NCHUNKS,))],
    )(xt)
    return out.T


# speedup vs baseline: 29.6936x; 29.6936x over previous
import jax
import jax.numpy as jnp
from jax import lax
from jax.experimental import pallas as pl
from jax.experimental.pallas import tpu as pltpu
from jax.experimental.pallas import tpu_sc as plsc

_D = 16
_N = 1000000
_SEG = 62464
_CH = 7808
_NCH = _SEG // _CH
_TAIL = _N - 16 * _SEG


def _sc_body(in_hbm, out_hbm, b0, b1, tbuf, ls0, ls1, ss0, ss1):
    band = lax.axis_index("c") * 8
    s = lax.axis_index("s")
    col0 = s * _SEG
    bufs = (b0, b1)
    ld = (ls0, ls1)
    st = (ss0, ss1)

    def src(i):
        return in_hbm.at[pl.ds(band, 8), pl.ds(col0 + i * _CH, _CH)]

    def dst(i):
        return out_hbm.at[pl.ds(band, 8), pl.ds(col0 + i * _CH, _CH)]

    pltpu.make_async_copy(src(0), bufs[0], ld[0]).start()
    for i in range(_NCH):
        b = i % 2
        if i + 1 < _NCH:
            nb = (i + 1) % 2
            if i >= 1:
                pltpu.make_async_copy(bufs[nb], dst(i - 1), st[nb]).wait()
            pltpu.make_async_copy(src(i + 1), bufs[nb], ld[nb]).start()
        pltpu.make_async_copy(src(i), bufs[b], ld[b]).wait()
        pltpu.make_async_copy(bufs[b], dst(i), st[b]).start()
    lb = (_NCH - 1) % 2
    pltpu.make_async_copy(bufs[lb], dst(_NCH - 1), st[lb]).wait()

    @pl.when(s == 15)
    def _tail():
        tcol = 16 * _SEG
        pltpu.sync_copy(in_hbm.at[pl.ds(band, 8), pl.ds(tcol, _TAIL)], tbuf)
        pltpu.sync_copy(tbuf, out_hbm.at[pl.ds(band, 8), pl.ds(tcol, _TAIL)])


def kernel(c_embeddings):
    n, d = c_embeddings.shape
    xt = c_embeddings.T
    mesh = plsc.VectorSubcoreMesh(core_axis_name="c", subcore_axis_name="s")
    sc_copy = pl.kernel(
        _sc_body,
        out_type=jax.ShapeDtypeStruct((d, n), xt.dtype),
        mesh=mesh,
        scratch_types=[
            pltpu.VMEM((8, _CH), jnp.float32),
            pltpu.VMEM((8, _CH), jnp.float32),
            pltpu.VMEM((8, _TAIL), jnp.float32),
            pltpu.SemaphoreType.DMA,
            pltpu.SemaphoreType.DMA,
            pltpu.SemaphoreType.DMA,
            pltpu.SemaphoreType.DMA,
        ],
        compiler_params=pltpu.CompilerParams(use_tc_tiling_on_sc=True),
    )
    return sc_copy(xt).T


# speedup vs baseline: 49.4090x; 1.6640x over previous
import jax
import jax.numpy as jnp
from jax.experimental import pallas as pl
from jax.experimental.pallas import tpu as pltpu


def _copy_body(in_ref, out_ref):
    out_ref[...] = in_ref[...]


def kernel(c_embeddings):
    n, d = c_embeddings.shape
    xt = c_embeddings.T
    bc = 131072
    grid = (pl.cdiv(n, bc),)
    out = pl.pallas_call(
        _copy_body,
        out_shape=jax.ShapeDtypeStruct((d, n), xt.dtype),
        grid=grid,
        in_specs=[pl.BlockSpec((d, bc), lambda i: (0, i))],
        out_specs=pl.BlockSpec((d, bc), lambda i: (0, i)),
        compiler_params=pltpu.CompilerParams(dimension_semantics=("parallel",)),
    )(xt)
    return out.T


# speedup vs baseline: 50.2890x; 1.0178x over previous
import jax
import jax.numpy as jnp
from jax.experimental import pallas as pl
from jax.experimental.pallas import tpu as pltpu


def _copy_body(in_ref, out_ref):
    out_ref[...] = in_ref[...]


def kernel(c_embeddings):
    n, d = c_embeddings.shape
    xt = c_embeddings.T
    bc = 229376
    grid = (pl.cdiv(n, bc),)
    out = pl.pallas_call(
        _copy_body,
        out_shape=jax.ShapeDtypeStruct((d, n), xt.dtype),
        grid=grid,
        in_specs=[pl.BlockSpec((d, bc), lambda i: (0, i))],
        out_specs=pl.BlockSpec((d, bc), lambda i: (0, i)),
        compiler_params=pltpu.CompilerParams(dimension_semantics=("parallel",)),
    )(xt)
    return out.T
